# baseline (speedup 1.0000x reference)
import functools

import jax
import jax.numpy as jnp
from jax import lax
from jax.experimental import pallas as pl
from jax.experimental.pallas import tpu as pltpu
from jax.experimental.pallas import tpu_sc as plsc

THR = 0.7
KEEP_MIN = 100000

B, C, H, W = 4, 19, 512, 512
PIX = B * H * W
NW = 32
HPT = H // (NW // B)
CR, CW = 8, 256
NCW = W // CW
NCH = (HPT // CR) * NCW
L = 16


def _sc_body(score_hbm, tgt_hbm, se_hbm, st_hbm, sbuf, tbuf, sebuf, stbuf,
             insem, outsem):
    wid = lax.axis_index("s") * 2 + lax.axis_index("c")
    b = wid // (NW // B)
    hbase = (wid % (NW // B)) * HPT

    def chunk_off(j):
        return hbase + (j // NCW) * CR, (j % NCW) * CW

    def in_copies(j, bank):
        h0, w0 = chunk_off(j)
        return (
            pltpu.make_async_copy(
                score_hbm.at[b, :, pl.ds(h0, CR), pl.ds(w0, CW)],
                sbuf.at[bank], insem.at[bank]),
            pltpu.make_async_copy(
                tgt_hbm.at[b, pl.ds(h0, CR), pl.ds(w0, CW)],
                tbuf.at[bank], insem.at[bank]),
        )

    def out_copies(j, bank):
        h0, w0 = chunk_off(j)
        return (
            pltpu.make_async_copy(
                sebuf.at[bank], se_hbm.at[b, pl.ds(h0, CR), pl.ds(w0, CW)],
                outsem.at[bank]),
            pltpu.make_async_copy(
                stbuf.at[bank], st_hbm.at[b, pl.ds(h0, CR), pl.ds(w0, CW)],
                outsem.at[bank]),
        )

    for cp in in_copies(0, 0):
        cp.start()

    for j in range(NCH):
        bank = j % 2
        for cp in in_copies(j, bank):
            cp.wait()
        if j + 1 < NCH:
            for cp in in_copies(j + 1, 1 - bank):
                cp.start()
        if j >= 2:
            for cp in out_copies(j - 2, bank):
                cp.wait()

        def group(i, carry, bank=bank):
            r = i // (CW // L)
            sl = pl.ds((i % (CW // L)) * L, L)
            t = tbuf[bank, r, sl]
            se0 = jnp.zeros((L,), jnp.float32)
            se1 = jnp.zeros((L,), jnp.float32)
            st = jnp.zeros((L,), jnp.float32)
            for c in range(C):
                v = sbuf[bank, c, r, sl]
                if c % 2 == 0:
                    se0 = se0 + jnp.exp(v)
                else:
                    se1 = se1 + jnp.exp(v)
                st = jnp.where(t == c, v, st)
            sebuf[bank, r, sl] = se0 + se1
            stbuf[bank, r, sl] = st
            return carry

        lax.fori_loop(0, CR * CW // L, group, 0)
        for cp in out_copies(j, bank):
            cp.start()

    for cp in out_copies(NCH - 2, (NCH - 2) % 2):
        cp.wait()
    for cp in out_copies(NCH - 1, (NCH - 1) % 2):
        cp.wait()


def _sc_pass(score, target):
    mesh = plsc.VectorSubcoreMesh(core_axis_name="c", subcore_axis_name="s")
    fn = functools.partial(
        pl.kernel,
        mesh=mesh,
        out_type=[
            jax.ShapeDtypeStruct((B, H, W), jnp.float32),
            jax.ShapeDtypeStruct((B, H, W), jnp.float32),
        ],
        scratch_types=[
            pltpu.VMEM((2, C, CR, CW), jnp.float32),
            pltpu.VMEM((2, CR, CW), jnp.int32),
            pltpu.VMEM((2, CR, CW), jnp.float32),
            pltpu.VMEM((2, CR, CW), jnp.float32),
            pltpu.SemaphoreType.DMA((2,)),
            pltpu.SemaphoreType.DMA((2,)),
        ],
    )(_sc_body)
    return fn(score, target)


def _tc_body(se_ref, st_ref, p_ref, nll_ref, stats_ref, acc_ref):
    i = pl.program_id(0)
    ni = pl.num_programs(0)

    @pl.when(i == 0)
    def _init():
        acc_ref[0] = jnp.float32(0.0)
        acc_ref[1] = jnp.float32(0.0)
        acc_ref[2] = jnp.float32(0.0)

    se = se_ref[0]
    st = st_ref[0]
    p = jnp.exp(st) / se
    nll = jnp.log(se) - st
    p_ref[0] = p
    nll_ref[0] = nll

    lt = p < THR
    acc_ref[0] += jnp.sum(jnp.where(p <= THR, 1.0, 0.0))
    acc_ref[1] += jnp.sum(jnp.where(lt, 1.0, 0.0))
    acc_ref[2] += jnp.sum(jnp.where(lt, nll, 0.0))

    @pl.when(i == ni - 1)
    def _fin():
        stats_ref[0] = acc_ref[0]
        stats_ref[1] = acc_ref[1]
        stats_ref[2] = acc_ref[2]


def _tc_pass(se, st):
    return pl.pallas_call(
        _tc_body,
        grid=(B,),
        in_specs=[
            pl.BlockSpec((1, H, W), lambda i: (i, 0, 0)),
            pl.BlockSpec((1, H, W), lambda i: (i, 0, 0)),
        ],
        out_specs=[
            pl.BlockSpec((1, H, W), lambda i: (i, 0, 0)),
            pl.BlockSpec((1, H, W), lambda i: (i, 0, 0)),
            pl.BlockSpec(memory_space=pltpu.SMEM, index_map=lambda i: (0,)),
        ],
        out_shape=[
            jax.ShapeDtypeStruct((B, H, W), jnp.float32),
            jax.ShapeDtypeStruct((B, H, W), jnp.float32),
            jax.ShapeDtypeStruct((3,), jnp.float32),
        ],
        scratch_shapes=[pltpu.SMEM((3,), jnp.float32)],
    )(se, st)


def kernel(score, target):
    se, st = _sc_pass(score, target)
    p, nll, stats = _tc_pass(se, st)
    cnt_le, cnt_lt, loss_sum = stats[0], stats[1], stats[2]

    def common(_):
        return loss_sum / cnt_lt

    def rare(_):
        ps = jnp.sort(p.reshape(-1))
        thr = jnp.maximum(ps[KEEP_MIN - 1], jnp.float32(THR))
        keep = p < thr
        tot = jnp.sum(jnp.where(keep, nll, 0.0))
        cnt = jnp.sum(keep).astype(jnp.float32)
        return tot / cnt

    return lax.cond(cnt_le >= KEEP_MIN, common, rare, None)


# speedup vs baseline: 1.3305x; 1.3305x over previous
import functools

import jax
import jax.numpy as jnp
from jax import lax
from jax.experimental import pallas as pl
from jax.experimental.pallas import tpu as pltpu
from jax.experimental.pallas import tpu_sc as plsc

THR = 0.7
KEEP_MIN = 100000

B, C, H, W = 4, 19, 512, 512
NW = 32
B_SC0 = 2
TPI = NW // (B - B_SC0)
HPT = H // TPI
CR, CW = 8, 256
NCW = W // CW
NCH = (HPT // CR) * NCW
L = 16


def _sc_body(score_hbm, tgt_hbm, se_hbm, st_hbm, sbuf, tbuf, sebuf, stbuf,
             insem, outsem):
    wid = lax.axis_index("s") * 2 + lax.axis_index("c")
    bo = wid // TPI
    b = B_SC0 + bo
    hbase = (wid % TPI) * HPT

    def chunk_off(j):
        return hbase + (j // NCW) * CR, (j % NCW) * CW

    def in_copies(j, bank):
        h0, w0 = chunk_off(j)
        return (
            pltpu.make_async_copy(
                score_hbm.at[b, :, pl.ds(h0, CR), pl.ds(w0, CW)],
                sbuf.at[bank], insem.at[bank]),
            pltpu.make_async_copy(
                tgt_hbm.at[b, pl.ds(h0, CR), pl.ds(w0, CW)],
                tbuf.at[bank], insem.at[bank]),
        )

    def out_copies(j, bank):
        h0, w0 = chunk_off(j)
        return (
            pltpu.make_async_copy(
                sebuf.at[bank], se_hbm.at[bo, pl.ds(h0, CR), pl.ds(w0, CW)],
                outsem.at[bank]),
            pltpu.make_async_copy(
                stbuf.at[bank], st_hbm.at[bo, pl.ds(h0, CR), pl.ds(w0, CW)],
                outsem.at[bank]),
        )

    for cp in in_copies(0, 0):
        cp.start()

    for j in range(NCH):
        bank = j % 2
        for cp in in_copies(j, bank):
            cp.wait()
        if j + 1 < NCH:
            for cp in in_copies(j + 1, 1 - bank):
                cp.start()
        if j >= 2:
            for cp in out_copies(j - 2, bank):
                cp.wait()

        def group(i, carry, bank=bank):
            r = i // (CW // L)
            sl = pl.ds((i % (CW // L)) * L, L)
            t = tbuf[bank, r, sl]
            se0 = jnp.zeros((L,), jnp.float32)
            se1 = jnp.zeros((L,), jnp.float32)
            st = jnp.zeros((L,), jnp.float32)
            for c in range(C):
                v = sbuf[bank, c, r, sl]
                if c % 2 == 0:
                    se0 = se0 + jnp.exp(v)
                else:
                    se1 = se1 + jnp.exp(v)
                st = jnp.where(t == c, v, st)
            sebuf[bank, r, sl] = se0 + se1
            stbuf[bank, r, sl] = st
            return carry

        lax.fori_loop(0, CR * CW // L, group, 0)
        for cp in out_copies(j, bank):
            cp.start()

    for cp in out_copies(NCH - 2, (NCH - 2) % 2):
        cp.wait()
    for cp in out_copies(NCH - 1, (NCH - 1) % 2):
        cp.wait()


def _sc_pass(score, target):
    mesh = plsc.VectorSubcoreMesh(core_axis_name="c", subcore_axis_name="s")
    nb = B - B_SC0
    fn = functools.partial(
        pl.kernel,
        mesh=mesh,
        out_type=[
            jax.ShapeDtypeStruct((nb, H, W), jnp.float32),
            jax.ShapeDtypeStruct((nb, H, W), jnp.float32),
        ],
        scratch_types=[
            pltpu.VMEM((2, C, CR, CW), jnp.float32),
            pltpu.VMEM((2, CR, CW), jnp.int32),
            pltpu.VMEM((2, CR, CW), jnp.float32),
            pltpu.VMEM((2, CR, CW), jnp.float32),
            pltpu.SemaphoreType.DMA((2,)),
            pltpu.SemaphoreType.DMA((2,)),
        ],
    )(_sc_body)
    return fn(score, target)


def _stats_init(acc_ref):
    acc_ref[0] = jnp.float32(0.0)
    acc_ref[1] = jnp.float32(0.0)
    acc_ref[2] = jnp.float32(0.0)


def _stats_accum(acc_ref, p, nll):
    lt = p < THR
    acc_ref[0] += jnp.sum(jnp.where(p <= THR, 1.0, 0.0))
    acc_ref[1] += jnp.sum(jnp.where(lt, 1.0, 0.0))
    acc_ref[2] += jnp.sum(jnp.where(lt, nll, 0.0))


def _tca_body(score_ref, tgt_ref, p_ref, nll_ref, stats_ref, acc_ref):
    b = pl.program_id(0)
    i = pl.program_id(1)
    first = jnp.logical_and(b == 0, i == 0)
    last = jnp.logical_and(b == pl.num_programs(0) - 1,
                           i == pl.num_programs(1) - 1)

    @pl.when(first)
    def _init():
        _stats_init(acc_ref)

    s = score_ref[0]
    t = tgt_ref[0]
    m = jnp.max(s, axis=0)
    e = jnp.exp(s - m[None])
    se = jnp.sum(e, axis=0)
    cls = lax.broadcasted_iota(jnp.int32, s.shape, 0)
    onehot = cls == t[None]
    e_t = jnp.sum(jnp.where(onehot, e, 0.0), axis=0)
    s_t = jnp.sum(jnp.where(onehot, s, 0.0), axis=0)
    p = e_t / se
    nll = jnp.log(se) - (s_t - m)
    p_ref[0] = p
    nll_ref[0] = nll
    _stats_accum(acc_ref, p, nll)

    @pl.when(last)
    def _fin():
        stats_ref[0] = acc_ref[0]
        stats_ref[1] = acc_ref[1]
        stats_ref[2] = acc_ref[2]


def _tc_pass_a(score, target):
    RH = 64
    nb = B_SC0
    return pl.pallas_call(
        _tca_body,
        grid=(nb, H // RH),
        in_specs=[
            pl.BlockSpec((1, C, RH, W), lambda b, i: (b, 0, i, 0)),
            pl.BlockSpec((1, RH, W), lambda b, i: (b, i, 0)),
        ],
        out_specs=[
            pl.BlockSpec((1, RH, W), lambda b, i: (b, i, 0)),
            pl.BlockSpec((1, RH, W), lambda b, i: (b, i, 0)),
            pl.BlockSpec(memory_space=pltpu.SMEM, index_map=lambda b, i: (0,)),
        ],
        out_shape=[
            jax.ShapeDtypeStruct((nb, H, W), jnp.float32),
            jax.ShapeDtypeStruct((nb, H, W), jnp.float32),
            jax.ShapeDtypeStruct((3,), jnp.float32),
        ],
        scratch_shapes=[pltpu.SMEM((3,), jnp.float32)],
    )(score, target)


def _tcb_body(se_ref, st_ref, p_ref, nll_ref, stats_ref, acc_ref):
    i = pl.program_id(0)

    @pl.when(i == 0)
    def _init():
        _stats_init(acc_ref)

    se = se_ref[0]
    st = st_ref[0]
    p = jnp.exp(st) / se
    nll = jnp.log(se) - st
    p_ref[0] = p
    nll_ref[0] = nll
    _stats_accum(acc_ref, p, nll)

    @pl.when(i == pl.num_programs(0) - 1)
    def _fin():
        stats_ref[0] = acc_ref[0]
        stats_ref[1] = acc_ref[1]
        stats_ref[2] = acc_ref[2]


def _tc_pass_b(se, st):
    nb = B - B_SC0
    return pl.pallas_call(
        _tcb_body,
        grid=(nb,),
        in_specs=[
            pl.BlockSpec((1, H, W), lambda i: (i, 0, 0)),
            pl.BlockSpec((1, H, W), lambda i: (i, 0, 0)),
        ],
        out_specs=[
            pl.BlockSpec((1, H, W), lambda i: (i, 0, 0)),
            pl.BlockSpec((1, H, W), lambda i: (i, 0, 0)),
            pl.BlockSpec(memory_space=pltpu.SMEM, index_map=lambda i: (0,)),
        ],
        out_shape=[
            jax.ShapeDtypeStruct((nb, H, W), jnp.float32),
            jax.ShapeDtypeStruct((nb, H, W), jnp.float32),
            jax.ShapeDtypeStruct((3,), jnp.float32),
        ],
        scratch_shapes=[pltpu.SMEM((3,), jnp.float32)],
    )(se, st)


def kernel(score, target):
    se23, st23 = _sc_pass(score, target)
    p01, nll01, stats01 = _tc_pass_a(score, target)
    p23, nll23, stats23 = _tc_pass_b(se23, st23)
    stats = stats01 + stats23
    cnt_le, cnt_lt, loss_sum = stats[0], stats[1], stats[2]

    def common(_):
        return loss_sum / cnt_lt

    def rare(_):
        p = jnp.concatenate([p01, p23], axis=0)
        nll = jnp.concatenate([nll01, nll23], axis=0)
        ps = jnp.sort(p.reshape(-1))
        thr = jnp.maximum(ps[KEEP_MIN - 1], jnp.float32(THR))
        keep = p < thr
        tot = jnp.sum(jnp.where(keep, nll, 0.0))
        cnt = jnp.sum(keep).astype(jnp.float32)
        return tot / cnt

    return lax.cond(cnt_le >= KEEP_MIN, common, rare, None)


# speedup vs baseline: 1.3350x; 1.0033x over previous
import functools

import jax
import jax.numpy as jnp
from jax import lax
from jax.experimental import pallas as pl
from jax.experimental.pallas import tpu as pltpu
from jax.experimental.pallas import tpu_sc as plsc

THR = 0.7
KEEP_MIN = 100000

B, C, H, W = 4, 19, 512, 512
NW = 32
B_SC0 = 2
TPI = NW // (B - B_SC0)
HPT = H // TPI
CR, CW = 8, 256
NCW = W // CW
NCH = (HPT // CR) * NCW
L = 16


def _sc_body(score_hbm, tgt_hbm, se_hbm, st_hbm, sbuf, tbuf, sebuf, stbuf,
             insem, outsem):
    wid = lax.axis_index("s") * 2 + lax.axis_index("c")
    bo = wid // TPI
    b = B_SC0 + bo
    hbase = (wid % TPI) * HPT

    def chunk_off(j):
        return hbase + (j // NCW) * CR, (j % NCW) * CW

    def in_copies(j, bank):
        h0, w0 = chunk_off(j)
        return (
            pltpu.make_async_copy(
                score_hbm.at[b, :, pl.ds(h0, CR), pl.ds(w0, CW)],
                sbuf.at[bank], insem.at[bank]),
            pltpu.make_async_copy(
                tgt_hbm.at[b, pl.ds(h0, CR), pl.ds(w0, CW)],
                tbuf.at[bank], insem.at[bank]),
        )

    def out_copies(j, bank):
        h0, w0 = chunk_off(j)
        return (
            pltpu.make_async_copy(
                sebuf.at[bank], se_hbm.at[bo, pl.ds(h0, CR), pl.ds(w0, CW)],
                outsem.at[bank]),
            pltpu.make_async_copy(
                stbuf.at[bank], st_hbm.at[bo, pl.ds(h0, CR), pl.ds(w0, CW)],
                outsem.at[bank]),
        )

    for cp in in_copies(0, 0):
        cp.start()

    for j in range(NCH):
        bank = j % 2
        for cp in in_copies(j, bank):
            cp.wait()
        if j + 1 < NCH:
            for cp in in_copies(j + 1, 1 - bank):
                cp.start()
        if j >= 2:
            for cp in out_copies(j - 2, bank):
                cp.wait()

        def group(i, carry, bank=bank):
            r = i // (CW // L)
            sl = pl.ds((i % (CW // L)) * L, L)
            t = tbuf[bank, r, sl]
            se0 = jnp.zeros((L,), jnp.float32)
            se1 = jnp.zeros((L,), jnp.float32)
            st = jnp.zeros((L,), jnp.float32)
            for c in range(C):
                v = sbuf[bank, c, r, sl]
                if c % 2 == 0:
                    se0 = se0 + jnp.exp(v)
                else:
                    se1 = se1 + jnp.exp(v)
                st = jnp.where(t == c, v, st)
            sebuf[bank, r, sl] = se0 + se1
            stbuf[bank, r, sl] = st
            return carry

        lax.fori_loop(0, CR * CW // L, group, 0)
        for cp in out_copies(j, bank):
            cp.start()

    for cp in out_copies(NCH - 2, (NCH - 2) % 2):
        cp.wait()
    for cp in out_copies(NCH - 1, (NCH - 1) % 2):
        cp.wait()


def _sc_pass(score, target):
    mesh = plsc.VectorSubcoreMesh(core_axis_name="c", subcore_axis_name="s")
    nb = B - B_SC0
    fn = functools.partial(
        pl.kernel,
        mesh=mesh,
        out_type=[
            jax.ShapeDtypeStruct((nb, H, W), jnp.float32),
            jax.ShapeDtypeStruct((nb, H, W), jnp.float32),
        ],
        scratch_types=[
            pltpu.VMEM((2, C, CR, CW), jnp.float32),
            pltpu.VMEM((2, CR, CW), jnp.int32),
            pltpu.VMEM((2, CR, CW), jnp.float32),
            pltpu.VMEM((2, CR, CW), jnp.float32),
            pltpu.SemaphoreType.DMA((2,)),
            pltpu.SemaphoreType.DMA((2,)),
        ],
    )(_sc_body)
    return fn(score, target)


def _stats_init(acc_ref):
    acc_ref[0] = jnp.float32(0.0)
    acc_ref[1] = jnp.float32(0.0)
    acc_ref[2] = jnp.float32(0.0)


def _stats_accum(acc_ref, p, nll):
    lt = p < THR
    acc_ref[0] += jnp.sum(jnp.where(p <= THR, 1.0, 0.0))
    acc_ref[1] += jnp.sum(jnp.where(lt, 1.0, 0.0))
    acc_ref[2] += jnp.sum(jnp.where(lt, nll, 0.0))


def _tc_softmax_math(s, t):
    m = jnp.max(s, axis=0)
    e = jnp.exp(s - m[None])
    se = jnp.sum(e, axis=0)
    cls = lax.broadcasted_iota(jnp.int32, s.shape, 0)
    onehot = cls == t[None]
    e_t = jnp.sum(jnp.where(onehot, e, 0.0), axis=0)
    s_t = jnp.sum(jnp.where(onehot, s, 0.0), axis=0)
    p = e_t / se
    nll = jnp.log(se) - (s_t - m)
    return p, nll


def _tca_body(emit_pnll, score_ref, tgt_ref, *refs):
    if emit_pnll:
        p_ref, nll_ref, stats_ref, acc_ref = refs
    else:
        stats_ref, acc_ref = refs
    b = pl.program_id(0)
    i = pl.program_id(1)
    first = jnp.logical_and(b == 0, i == 0)
    last = jnp.logical_and(b == pl.num_programs(0) - 1,
                           i == pl.num_programs(1) - 1)

    @pl.when(first)
    def _init():
        _stats_init(acc_ref)

    p, nll = _tc_softmax_math(score_ref[0], tgt_ref[0])
    if emit_pnll:
        p_ref[0] = p
        nll_ref[0] = nll
    _stats_accum(acc_ref, p, nll)

    @pl.when(last)
    def _fin():
        stats_ref[0] = acc_ref[0]
        stats_ref[1] = acc_ref[1]
        stats_ref[2] = acc_ref[2]


def _tc_pass_a(score, target, emit_pnll):
    RH = 64
    nb = B_SC0
    pnll_specs = [
        pl.BlockSpec((1, RH, W), lambda b, i: (b, i, 0)),
        pl.BlockSpec((1, RH, W), lambda b, i: (b, i, 0)),
    ]
    pnll_shapes = [
        jax.ShapeDtypeStruct((nb, H, W), jnp.float32),
        jax.ShapeDtypeStruct((nb, H, W), jnp.float32),
    ]
    return pl.pallas_call(
        functools.partial(_tca_body, emit_pnll),
        grid=(nb, H // RH),
        in_specs=[
            pl.BlockSpec((1, C, RH, W), lambda b, i: (b, 0, i, 0)),
            pl.BlockSpec((1, RH, W), lambda b, i: (b, i, 0)),
        ],
        out_specs=(pnll_specs if emit_pnll else []) + [
            pl.BlockSpec(memory_space=pltpu.SMEM, index_map=lambda b, i: (0,)),
        ],
        out_shape=(pnll_shapes if emit_pnll else []) + [
            jax.ShapeDtypeStruct((3,), jnp.float32),
        ],
        scratch_shapes=[pltpu.SMEM((3,), jnp.float32)],
    )(score, target)


def _tcb_body(emit_pnll, se_ref, st_ref, *refs):
    if emit_pnll:
        p_ref, nll_ref, stats_ref, acc_ref = refs
    else:
        stats_ref, acc_ref = refs
    i = pl.program_id(0)

    @pl.when(i == 0)
    def _init():
        _stats_init(acc_ref)

    se = se_ref[0]
    st = st_ref[0]
    p = jnp.exp(st) / se
    nll = jnp.log(se) - st
    if emit_pnll:
        p_ref[0] = p
        nll_ref[0] = nll
    _stats_accum(acc_ref, p, nll)

    @pl.when(i == pl.num_programs(0) - 1)
    def _fin():
        stats_ref[0] = acc_ref[0]
        stats_ref[1] = acc_ref[1]
        stats_ref[2] = acc_ref[2]


def _tc_pass_b(se, st, emit_pnll):
    nb = B - B_SC0
    pnll_specs = [
        pl.BlockSpec((1, H, W), lambda i: (i, 0, 0)),
        pl.BlockSpec((1, H, W), lambda i: (i, 0, 0)),
    ]
    pnll_shapes = [
        jax.ShapeDtypeStruct((nb, H, W), jnp.float32),
        jax.ShapeDtypeStruct((nb, H, W), jnp.float32),
    ]
    return pl.pallas_call(
        functools.partial(_tcb_body, emit_pnll),
        grid=(nb,),
        in_specs=[
            pl.BlockSpec((1, H, W), lambda i: (i, 0, 0)),
            pl.BlockSpec((1, H, W), lambda i: (i, 0, 0)),
        ],
        out_specs=(pnll_specs if emit_pnll else []) + [
            pl.BlockSpec(memory_space=pltpu.SMEM, index_map=lambda i: (0,)),
        ],
        out_shape=(pnll_shapes if emit_pnll else []) + [
            jax.ShapeDtypeStruct((3,), jnp.float32),
        ],
        scratch_shapes=[pltpu.SMEM((3,), jnp.float32)],
    )(se, st)


def kernel(score, target):
    se23, st23 = _sc_pass(score, target)
    stats01 = _tc_pass_a(score, target, emit_pnll=False)[0]
    stats23 = _tc_pass_b(se23, st23, emit_pnll=False)[0]
    stats = stats01 + stats23
    cnt_le, cnt_lt, loss_sum = stats[0], stats[1], stats[2]

    def common(_):
        return loss_sum / cnt_lt

    def rare(_):
        p01, nll01, _ = _tc_pass_a(score, target, emit_pnll=True)
        p23, nll23, _ = _tc_pass_b(se23, st23, emit_pnll=True)
        p = jnp.concatenate([p01, p23], axis=0)
        nll = jnp.concatenate([nll01, nll23], axis=0)
        ps = jnp.sort(p.reshape(-1))
        thr = jnp.maximum(ps[KEEP_MIN - 1], jnp.float32(THR))
        keep = p < thr
        tot = jnp.sum(jnp.where(keep, nll, 0.0))
        cnt = jnp.sum(keep).astype(jnp.float32)
        return tot / cnt

    return lax.cond(cnt_le >= KEEP_MIN, common, rare, None)
